# baseline (speedup 1.0000x reference)
import functools

import jax
import jax.numpy as jnp
from jax import lax
from jax.experimental import pallas as pl
from jax.experimental.pallas import tpu as pltpu
from jax.experimental.pallas import tpu_sc as plsc

_B = 4096
_HB = 2048
_D = 128
_H1 = 1024
_H2 = 512
_H3 = 128
_BB = 1024
_PK = 40


def _sc_gather(uid, zid, utab, ztab):
    info = plsc.get_sparse_core_info()
    nc, ns = info.num_cores, info.num_subcores
    nw = nc * ns
    bpw = _HB // nw

    mesh = plsc.VectorSubcoreMesh(core_axis_name="c", subcore_axis_name="s")
    out_t = jax.ShapeDtypeStruct((_HB, 2 * _D), jnp.float32)
    scratch = (
        [pltpu.VMEM((bpw,), jnp.int32) for _ in range(2)]
        + [pltpu.VMEM((bpw, _D), jnp.float32) for _ in range(2)]
        + [pltpu.SemaphoreType.DMA, pltpu.SemaphoreType.DMA]
    )

    nch = 4
    ch = bpw // nch

    @functools.partial(pl.kernel, mesh=mesh, out_type=out_t,
                       scratch_types=scratch)
    def gather(uid_h, zid_h, utab_h, ztab_h, o_h,
               ui_v, zi_v, ur_v, zr_v, sem, wsem):
        wid = lax.axis_index("s") * nc + lax.axis_index("c")
        base = wid * bpw
        sl = pl.ds(base, bpw)
        iu = pltpu.async_copy(uid_h.at[sl], ui_v, sem)
        iz = pltpu.async_copy(zid_h.at[sl], zi_v, sem)
        iu.wait()
        iz.wait()
        gs = []
        for c in range(nch):
            cs = pl.ds(c * ch, ch)
            gs.append(pltpu.async_copy(utab_h.at[ui_v.at[cs]], ur_v.at[cs],
                                       sem))
            gs.append(pltpu.async_copy(ztab_h.at[zi_v.at[cs]], zr_v.at[cs],
                                       sem))
        ws = []
        for c in range(nch):
            osl = pl.ds(base + c * ch, ch)
            cs = pl.ds(c * ch, ch)
            gs[2 * c].wait()
            ws.append(pltpu.async_copy(ur_v.at[cs],
                                       o_h.at[osl, pl.ds(0, _D)], wsem))
            gs[2 * c + 1].wait()
            ws.append(pltpu.async_copy(zr_v.at[cs],
                                       o_h.at[osl, pl.ds(_D, _D)], wsem))
        for w in ws:
            w.wait()

    return gather(uid, zid, utab, ztab)


def _mlp_body(uz, g, a, o, gt, at, ot, w1u, w1m, w1z, b1, w2, b2, w3, b3,
              out, p_s):
    bf = jnp.bfloat16

    @pl.when(pl.program_id(0) == 0)
    def _prep():
        p_s[...] = jnp.zeros((_PK, _H1), dtype=bf)
        p_s[0:2, :] = jnp.dot(
            gt[...].astype(bf), w1m[0:_D, :],
            preferred_element_type=jnp.float32).astype(bf)
        p_s[8:15, :] = jnp.dot(
            at[...].astype(bf), w1m[_D:2 * _D, :],
            preferred_element_type=jnp.float32).astype(bf)
        p_s[16:37, :] = jnp.dot(
            ot[...].astype(bf), w1m[2 * _D:3 * _D, :],
            preferred_element_type=jnp.float32).astype(bf)

    gi = g[0, 0, :]
    ai = a[0, 0, :]
    oi = o[0, 0, :]
    i40 = lax.broadcasted_iota(jnp.int32, (_BB, _PK), 1)
    coh = ((gi[:, None] == i40) | (ai[:, None] + 8 == i40)
           | (oi[:, None] + 16 == i40)).astype(bf)
    uzb = uz[...].astype(bf)
    h = (jnp.dot(uzb[:, :_D], w1u[...],
                 preferred_element_type=jnp.float32)
         + jnp.dot(uzb[:, _D:], w1z[...],
                   preferred_element_type=jnp.float32)
         + jnp.dot(coh, p_s[...], preferred_element_type=jnp.float32)
         + b1[...])
    h = jnp.maximum(h, 0.0).astype(bf)
    h = jnp.dot(h, w2[...], preferred_element_type=jnp.float32) + b2[...]
    h = jnp.maximum(h, 0.0).astype(bf)
    h = jnp.dot(h, w3[...], preferred_element_type=jnp.float32) + b3[...]
    h = jnp.maximum(h, 0.0)
    ss = jnp.sum(h * h, axis=1, keepdims=True)
    out[...] = h * lax.rsqrt(jnp.maximum(ss, 1e-24))


def _mlp(uz, gid, aid, oid, gtab, atab, otab, w1, b1, w2, b2, w3, b3):
    nblk = _HB // _BB
    bf = jnp.bfloat16
    const = lambda i: (0, 0)
    w1b = w1.astype(bf)
    return pl.pallas_call(
        _mlp_body,
        grid=(nblk,),
        in_specs=[
            pl.BlockSpec((_BB, 2 * _D), lambda i: (i, 0)),
            pl.BlockSpec((1, 1, _BB), lambda i: (i, 0, 0)),
            pl.BlockSpec((1, 1, _BB), lambda i: (i, 0, 0)),
            pl.BlockSpec((1, 1, _BB), lambda i: (i, 0, 0)),
            pl.BlockSpec((2, _D), const),
            pl.BlockSpec((7, _D), const),
            pl.BlockSpec((21, _D), const),
            pl.BlockSpec((_D, _H1), const),
            pl.BlockSpec((3 * _D, _H1), const),
            pl.BlockSpec((_D, _H1), const),
            pl.BlockSpec((1, _H1), const),
            pl.BlockSpec((_H1, _H2), const),
            pl.BlockSpec((1, _H2), const),
            pl.BlockSpec((_H2, _H3), const),
            pl.BlockSpec((1, _H3), const),
        ],
        out_specs=pl.BlockSpec((_BB, _D), lambda i: (i, 0)),
        out_shape=jax.ShapeDtypeStruct((_HB, _D), jnp.float32),
        scratch_shapes=[
            pltpu.VMEM((_PK, _H1), jnp.bfloat16),
        ],
    )(uz, gid.reshape(nblk, 1, _BB), aid.reshape(nblk, 1, _BB),
      oid.reshape(nblk, 1, _BB), gtab, atab, otab,
      w1b[0:_D], w1b[_D:4 * _D], w1b[4 * _D:5 * _D], b1.reshape(1, _H1),
      w2.astype(bf), b2.reshape(1, _H2), w3.astype(bf), b3.reshape(1, _H3))


def kernel(user_id, gender, age, occupation, zip_id, user_tab, gender_tab,
           age_tab, occ_tab, zip_tab, W1, b1, W2, b2, W3, b3):
    uid = user_id.astype(jnp.int32)
    gid = gender.astype(jnp.int32)
    aid = age.astype(jnp.int32)
    oid = occupation.astype(jnp.int32)
    zid = zip_id.astype(jnp.int32)
    uz_a = _sc_gather(uid[:_HB], zid[:_HB], user_tab, zip_tab)
    uz_b = _sc_gather(uid[_HB:], zid[_HB:], user_tab, zip_tab)
    out_a = _mlp(uz_a, gid[:_HB], aid[:_HB], oid[:_HB], gender_tab, age_tab,
                 occ_tab, W1, b1, W2, b2, W3, b3)
    out_b = _mlp(uz_b, gid[_HB:], aid[_HB:], oid[_HB:], gender_tab, age_tab,
                 occ_tab, W1, b1, W2, b2, W3, b3)
    return jnp.concatenate([out_a, out_b], axis=0)


# speedup vs baseline: 1.2176x; 1.2176x over previous
import functools

import jax
import jax.numpy as jnp
from jax import lax
from jax.experimental import pallas as pl
from jax.experimental.pallas import tpu as pltpu
from jax.experimental.pallas import tpu_sc as plsc

_B = 4096
_HB = _B
_D = 128
_H1 = 1024
_H2 = 512
_H3 = 128
_BB = 1024
_PK = 40


def _sc_gather(uid, zid, utab, ztab):
    info = plsc.get_sparse_core_info()
    nc, ns = info.num_cores, info.num_subcores
    nw = nc * ns
    bpw = _HB // nw

    mesh = plsc.VectorSubcoreMesh(core_axis_name="c", subcore_axis_name="s")
    out_t = jax.ShapeDtypeStruct((_HB, 2 * _D), jnp.float32)
    scratch = (
        [pltpu.VMEM((bpw,), jnp.int32) for _ in range(2)]
        + [pltpu.VMEM((bpw, _D), jnp.float32) for _ in range(2)]
        + [pltpu.SemaphoreType.DMA, pltpu.SemaphoreType.DMA]
    )

    nch = 4
    ch = bpw // nch

    @functools.partial(pl.kernel, mesh=mesh, out_type=out_t,
                       scratch_types=scratch)
    def gather(uid_h, zid_h, utab_h, ztab_h, o_h,
               ui_v, zi_v, ur_v, zr_v, sem, wsem):
        wid = lax.axis_index("s") * nc + lax.axis_index("c")
        base = wid * bpw
        sl = pl.ds(base, bpw)
        iu = pltpu.async_copy(uid_h.at[sl], ui_v, sem)
        iz = pltpu.async_copy(zid_h.at[sl], zi_v, sem)
        iu.wait()
        iz.wait()
        gs = []
        for c in range(nch):
            cs = pl.ds(c * ch, ch)
            gs.append(pltpu.async_copy(utab_h.at[ui_v.at[cs]], ur_v.at[cs],
                                       sem))
            gs.append(pltpu.async_copy(ztab_h.at[zi_v.at[cs]], zr_v.at[cs],
                                       sem))
        ws = []
        for c in range(nch):
            osl = pl.ds(base + c * ch, ch)
            cs = pl.ds(c * ch, ch)
            gs[2 * c].wait()
            ws.append(pltpu.async_copy(ur_v.at[cs],
                                       o_h.at[osl, pl.ds(0, _D)], wsem))
            gs[2 * c + 1].wait()
            ws.append(pltpu.async_copy(zr_v.at[cs],
                                       o_h.at[osl, pl.ds(_D, _D)], wsem))
        for w in ws:
            w.wait()

    return gather(uid, zid, utab, ztab)


def _mlp_body(uz, g, a, o, gt, at, ot, w1u, w1m, w1z, b1, w2, b2, w3, b3,
              out, w1s):
    bf = jnp.bfloat16

    @pl.when(pl.program_id(0) == 0)
    def _prep():
        w1s[0:2 * _D, :] = jnp.concatenate([w1u[...], w1z[...]], axis=0)
        w1s[2 * _D:, :] = jnp.zeros((_PK, _H1), dtype=bf)
        w1s[2 * _D:2 * _D + 2, :] = jnp.dot(
            gt[...].astype(bf), w1m[0:_D, :],
            preferred_element_type=jnp.float32).astype(bf)
        w1s[2 * _D + 8:2 * _D + 15, :] = jnp.dot(
            at[...].astype(bf), w1m[_D:2 * _D, :],
            preferred_element_type=jnp.float32).astype(bf)
        w1s[2 * _D + 16:2 * _D + 37, :] = jnp.dot(
            ot[...].astype(bf), w1m[2 * _D:3 * _D, :],
            preferred_element_type=jnp.float32).astype(bf)
        w1s[2 * _D + 37:2 * _D + 38, :] = b1[...].astype(bf)

    gi = g[0, 0, :]
    ai = a[0, 0, :]
    oi = o[0, 0, :]
    i40 = lax.broadcasted_iota(jnp.int32, (_BB, _PK), 1)
    coh = ((gi[:, None] == i40) | (ai[:, None] + 8 == i40)
           | (oi[:, None] + 16 == i40) | (i40 == 37)).astype(bf)
    x2 = jnp.concatenate([uz[...].astype(bf), coh], axis=1)
    h = jnp.dot(x2, w1s[...], preferred_element_type=jnp.float32)
    h = jnp.maximum(h, 0.0).astype(bf)
    h = jnp.dot(h, w2[...], preferred_element_type=jnp.float32) + b2[...]
    h = jnp.maximum(h, 0.0).astype(bf)
    h = jnp.dot(h, w3[...], preferred_element_type=jnp.float32) + b3[...]
    h = jnp.maximum(h, 0.0)
    ss = jnp.sum(h * h, axis=1, keepdims=True)
    out[...] = h * lax.rsqrt(jnp.maximum(ss, 1e-24))


def _mlp(uz, gid, aid, oid, gtab, atab, otab, w1, b1, w2, b2, w3, b3):
    nblk = _HB // _BB
    bf = jnp.bfloat16
    const = lambda i: (0, 0)
    w1b = w1.astype(bf)
    return pl.pallas_call(
        _mlp_body,
        grid=(nblk,),
        in_specs=[
            pl.BlockSpec((_BB, 2 * _D), lambda i: (i, 0)),
            pl.BlockSpec((1, 1, _BB), lambda i: (i, 0, 0)),
            pl.BlockSpec((1, 1, _BB), lambda i: (i, 0, 0)),
            pl.BlockSpec((1, 1, _BB), lambda i: (i, 0, 0)),
            pl.BlockSpec((2, _D), const),
            pl.BlockSpec((7, _D), const),
            pl.BlockSpec((21, _D), const),
            pl.BlockSpec((_D, _H1), const),
            pl.BlockSpec((3 * _D, _H1), const),
            pl.BlockSpec((_D, _H1), const),
            pl.BlockSpec((1, _H1), const),
            pl.BlockSpec((_H1, _H2), const),
            pl.BlockSpec((1, _H2), const),
            pl.BlockSpec((_H2, _H3), const),
            pl.BlockSpec((1, _H3), const),
        ],
        out_specs=pl.BlockSpec((_BB, _D), lambda i: (i, 0)),
        out_shape=jax.ShapeDtypeStruct((_HB, _D), jnp.float32),
        scratch_shapes=[
            pltpu.VMEM((2 * _D + _PK, _H1), jnp.bfloat16),
        ],
    )(uz, gid.reshape(nblk, 1, _BB), aid.reshape(nblk, 1, _BB),
      oid.reshape(nblk, 1, _BB), gtab, atab, otab,
      w1b[0:_D], w1b[_D:4 * _D], w1b[4 * _D:5 * _D], b1.reshape(1, _H1),
      w2.astype(bf), b2.reshape(1, _H2), w3.astype(bf), b3.reshape(1, _H3))


def kernel(user_id, gender, age, occupation, zip_id, user_tab, gender_tab,
           age_tab, occ_tab, zip_tab, W1, b1, W2, b2, W3, b3):
    uid = user_id.astype(jnp.int32)
    gid = gender.astype(jnp.int32)
    aid = age.astype(jnp.int32)
    oid = occupation.astype(jnp.int32)
    zid = zip_id.astype(jnp.int32)
    uz = _sc_gather(uid, zid, user_tab, zip_tab)
    return _mlp(uz, gid, aid, oid, gender_tab, age_tab, occ_tab,
                W1, b1, W2, b2, W3, b3)


# speedup vs baseline: 1.2422x; 1.0202x over previous
import functools

import jax
import jax.numpy as jnp
from jax import lax
from jax.experimental import pallas as pl
from jax.experimental.pallas import tpu as pltpu
from jax.experimental.pallas import tpu_sc as plsc

_B = 4096
_HB = _B
_D = 128
_H1 = 1024
_H2 = 512
_H3 = 128
_BB = 2048
_PK = 40


def _sc_gather(uid, zid, utab, ztab):
    info = plsc.get_sparse_core_info()
    nc, ns = info.num_cores, info.num_subcores
    nw = nc * ns
    bpw = _HB // nw

    mesh = plsc.VectorSubcoreMesh(core_axis_name="c", subcore_axis_name="s")
    out_t = jax.ShapeDtypeStruct((_HB, 2 * _D), jnp.float32)
    scratch = (
        [pltpu.VMEM((bpw,), jnp.int32) for _ in range(2)]
        + [pltpu.VMEM((bpw, _D), jnp.float32) for _ in range(2)]
        + [pltpu.SemaphoreType.DMA, pltpu.SemaphoreType.DMA]
    )

    nch = 4
    ch = bpw // nch

    @functools.partial(pl.kernel, mesh=mesh, out_type=out_t,
                       scratch_types=scratch)
    def gather(uid_h, zid_h, utab_h, ztab_h, o_h,
               ui_v, zi_v, ur_v, zr_v, sem, wsem):
        wid = lax.axis_index("s") * nc + lax.axis_index("c")
        base = wid * bpw
        sl = pl.ds(base, bpw)
        iu = pltpu.async_copy(uid_h.at[sl], ui_v, sem)
        iz = pltpu.async_copy(zid_h.at[sl], zi_v, sem)
        iu.wait()
        iz.wait()
        gs = []
        for c in range(nch):
            cs = pl.ds(c * ch, ch)
            gs.append(pltpu.async_copy(utab_h.at[ui_v.at[cs]], ur_v.at[cs],
                                       sem))
            gs.append(pltpu.async_copy(ztab_h.at[zi_v.at[cs]], zr_v.at[cs],
                                       sem))
        ws = []
        for c in range(nch):
            osl = pl.ds(base + c * ch, ch)
            cs = pl.ds(c * ch, ch)
            gs[2 * c].wait()
            ws.append(pltpu.async_copy(ur_v.at[cs],
                                       o_h.at[osl, pl.ds(0, _D)], wsem))
            gs[2 * c + 1].wait()
            ws.append(pltpu.async_copy(zr_v.at[cs],
                                       o_h.at[osl, pl.ds(_D, _D)], wsem))
        for w in ws:
            w.wait()

    return gather(uid, zid, utab, ztab)


def _mlp_body(uz, g, a, o, gt, at, ot, w1u, w1m, w1z, b1, w2, b2, w3, b3,
              out, w1s):
    bf = jnp.bfloat16

    @pl.when(pl.program_id(0) == 0)
    def _prep():
        w1s[0:2 * _D, :] = jnp.concatenate([w1u[...], w1z[...]], axis=0)
        w1s[2 * _D:, :] = jnp.zeros((_PK, _H1), dtype=bf)
        w1s[2 * _D:2 * _D + 2, :] = jnp.dot(
            gt[...].astype(bf), w1m[0:_D, :],
            preferred_element_type=jnp.float32).astype(bf)
        w1s[2 * _D + 8:2 * _D + 15, :] = jnp.dot(
            at[...].astype(bf), w1m[_D:2 * _D, :],
            preferred_element_type=jnp.float32).astype(bf)
        w1s[2 * _D + 16:2 * _D + 37, :] = jnp.dot(
            ot[...].astype(bf), w1m[2 * _D:3 * _D, :],
            preferred_element_type=jnp.float32).astype(bf)
        w1s[2 * _D + 37:2 * _D + 38, :] = b1[...].astype(bf)

    gi = g[0, 0, :]
    ai = a[0, 0, :]
    oi = o[0, 0, :]
    i40 = lax.broadcasted_iota(jnp.int32, (_BB, _PK), 1)
    coh = ((gi[:, None] == i40) | (ai[:, None] + 8 == i40)
           | (oi[:, None] + 16 == i40) | (i40 == 37)).astype(bf)
    x2 = jnp.concatenate([uz[...].astype(bf), coh], axis=1)
    h = jnp.dot(x2, w1s[...], preferred_element_type=jnp.float32)
    h = jnp.maximum(h, 0.0).astype(bf)
    h = jnp.dot(h, w2[...], preferred_element_type=jnp.float32) + b2[...]
    h = jnp.maximum(h, 0.0).astype(bf)
    h = jnp.dot(h, w3[...], preferred_element_type=jnp.float32) + b3[...]
    h = jnp.maximum(h, 0.0)
    ss = jnp.sum(h * h, axis=1, keepdims=True)
    out[...] = h * lax.rsqrt(jnp.maximum(ss, 1e-24))


def _mlp(uz, gid, aid, oid, gtab, atab, otab, w1, b1, w2, b2, w3, b3):
    nblk = _HB // _BB
    bf = jnp.bfloat16
    const = lambda i: (0, 0)
    w1b = w1.astype(bf)
    return pl.pallas_call(
        _mlp_body,
        grid=(nblk,),
        in_specs=[
            pl.BlockSpec((_BB, 2 * _D), lambda i: (i, 0)),
            pl.BlockSpec((1, 1, _BB), lambda i: (i, 0, 0)),
            pl.BlockSpec((1, 1, _BB), lambda i: (i, 0, 0)),
            pl.BlockSpec((1, 1, _BB), lambda i: (i, 0, 0)),
            pl.BlockSpec((2, _D), const),
            pl.BlockSpec((7, _D), const),
            pl.BlockSpec((21, _D), const),
            pl.BlockSpec((_D, _H1), const),
            pl.BlockSpec((3 * _D, _H1), const),
            pl.BlockSpec((_D, _H1), const),
            pl.BlockSpec((1, _H1), const),
            pl.BlockSpec((_H1, _H2), const),
            pl.BlockSpec((1, _H2), const),
            pl.BlockSpec((_H2, _H3), const),
            pl.BlockSpec((1, _H3), const),
        ],
        out_specs=pl.BlockSpec((_BB, _D), lambda i: (i, 0)),
        out_shape=jax.ShapeDtypeStruct((_HB, _D), jnp.float32),
        scratch_shapes=[
            pltpu.VMEM((2 * _D + _PK, _H1), jnp.bfloat16),
        ],
    )(uz, gid.reshape(nblk, 1, _BB), aid.reshape(nblk, 1, _BB),
      oid.reshape(nblk, 1, _BB), gtab, atab, otab,
      w1b[0:_D], w1b[_D:4 * _D], w1b[4 * _D:5 * _D], b1.reshape(1, _H1),
      w2.astype(bf), b2.reshape(1, _H2), w3.astype(bf), b3.reshape(1, _H3))


def kernel(user_id, gender, age, occupation, zip_id, user_tab, gender_tab,
           age_tab, occ_tab, zip_tab, W1, b1, W2, b2, W3, b3):
    uid = user_id.astype(jnp.int32)
    gid = gender.astype(jnp.int32)
    aid = age.astype(jnp.int32)
    oid = occupation.astype(jnp.int32)
    zid = zip_id.astype(jnp.int32)
    uz = _sc_gather(uid, zid, user_tab, zip_tab)
    return _mlp(uz, gid, aid, oid, gender_tab, age_tab, occ_tab,
                W1, b1, W2, b2, W3, b3)
